# baseline (speedup 1.0000x reference)
import contextlib
import functools

import numpy as np

import jax
import jax.numpy as jnp
from jax import lax
from jax.experimental import pallas as pl
from jax.experimental.pallas import tpu as pltpu
from jax.experimental.pallas import tpu_sc as plsc

_SWAP_RATE = 0.15
_B, _F = 16384, 100
_FP = 128
_NP = _B * _FP
_NW = 32
_ROWS_W = _B // _NW
_PER_W = _ROWS_W * _FP
_KMAX = 7840


@functools.lru_cache(maxsize=None)
def _swap_tables():
    try:
        dev = jax.local_devices(backend="cpu")[0]
        ctx = jax.default_device(dev)
    except Exception:
        ctx = contextlib.nullcontext()
    with ctx:
        k1, k2 = jax.random.split(jax.random.key(42))
        u1 = np.asarray(jax.random.uniform(k1, (_B, _F)))
        u2 = np.asarray(jax.random.uniform(k2, (_B, _F)))
    mask = u1 > np.float32(1.0 - _SWAP_RATE)
    l1 = np.floor(u2 * np.float32(_B)).astype(np.int64)
    n = _B * _F
    res = (l1 * (mask.astype(np.int64) * _F)).reshape(-1)
    idx = np.arange(n, dtype=np.int64) + res
    idx = np.where(idx >= n, idx - n, idx)
    pos_p = (np.arange(n, dtype=np.int64) // _F) * _FP + np.arange(n) % _F
    src_p = (idx // _F) * _FP + idx % _F
    delta = idx != np.arange(n, dtype=np.int64)
    src = np.empty((_NW, _KMAX), np.int32)
    dst = np.empty((_NW, _KMAX), np.int32)
    w_of = pos_p // _PER_W
    loc_of = pos_p % _PER_W
    full_src = np.arange(_NP, dtype=np.int64)
    full_src[pos_p] = src_p
    for w in range(_NW):
        sel = delta & (w_of == w)
        loc = loc_of[sel]
        s = src_p[sel]
        k = loc.size
        assert k <= _KMAX
        d = np.empty(_KMAX, np.int32)
        sfull = np.empty(_KMAX, np.int32)
        d[:k] = loc
        sfull[:k] = s
        npad = _KMAX - k
        t = np.arange(npad, dtype=np.int64) * 151 % (_ROWS_W * _F)
        pad_loc = (t // _F) * _FP + t % _F
        d[k:] = pad_loc
        sfull[k:] = full_src[w * _PER_W + pad_loc]
        dst[w] = d
        src[w] = sfull
    return src, dst


_SRC_TAB, _DST_TAB = _swap_tables()


@functools.lru_cache(maxsize=None)
def _make_sc_kernel():
    info = plsc.get_sparse_core_info()
    assert info.num_cores * info.num_subcores == _NW
    mesh = plsc.VectorSubcoreMesh(core_axis_name="c", subcore_axis_name="s")

    @functools.partial(
        pl.kernel,
        mesh=mesh,
        out_type=jax.ShapeDtypeStruct((_B, _F), jnp.float32),
        compiler_params=pltpu.CompilerParams(
            needs_layout_passes=False, use_tc_tiling_on_sc=True),
        scratch_types=[
            pltpu.VMEM((_ROWS_W, _F), jnp.float32),
            pltpu.VMEM((_KMAX,), jnp.int32),
            pltpu.VMEM((_KMAX,), jnp.int32),
            pltpu.VMEM((_KMAX,), jnp.float32),
            pltpu.SemaphoreType.DMA,
            pltpu.SemaphoreType.DMA,
        ],
    )
    def swap_k(x2_hbm, xf_hbm, src_hbm, dst_hbm, out_hbm, xv, srcv, dstv,
               valsv, sem0, sem1):
        wid = lax.axis_index("s") * info.num_cores + lax.axis_index("c")
        row0 = wid * _ROWS_W
        pltpu.sync_copy(src_hbm.at[wid], srcv)
        cp_g = pltpu.async_copy(xf_hbm.at[srcv], valsv, sem1)
        cp_x = pltpu.async_copy(x2_hbm.at[pl.ds(row0, _ROWS_W)], xv, sem0)
        pltpu.sync_copy(dst_hbm.at[wid], dstv)
        cp_x.wait()
        cp_g.wait()

        def body(k, carry):
            s = pl.ds(k * 16, 16)
            d = dstv[s]
            rows = lax.shift_right_logical(d, 7)
            cols = lax.bitwise_and(d, 127)
            plsc.store_scatter(xv, [rows, cols], valsv[s])
            return carry

        lax.fori_loop(0, _KMAX // 16, body, 0)
        pltpu.sync_copy(xv, out_hbm.at[pl.ds(row0, _ROWS_W)])

    return swap_k


def kernel(x):
    xp = jnp.pad(x, ((0, 0), (0, _FP - _F)))
    return _make_sc_kernel()(
        x, xp.reshape(-1), jnp.asarray(_SRC_TAB), jnp.asarray(_DST_TAB))


# speedup vs baseline: 1.0142x; 1.0142x over previous
import contextlib
import functools

import numpy as np

import jax
import jax.numpy as jnp
from jax import lax
from jax.experimental import pallas as pl
from jax.experimental.pallas import tpu as pltpu
from jax.experimental.pallas import tpu_sc as plsc

_SWAP_RATE = 0.15
_B, _F = 16384, 100
_FP = 128
_NP = _B * _FP
_NW = 32
_ROWS_W = _B // _NW
_PER_W = _ROWS_W * _FP
_KMAX = 7840


@functools.lru_cache(maxsize=None)
def _swap_tables():
    try:
        dev = jax.local_devices(backend="cpu")[0]
        ctx = jax.default_device(dev)
    except Exception:
        ctx = contextlib.nullcontext()
    with ctx:
        k1, k2 = jax.random.split(jax.random.key(42))
        u1 = np.asarray(jax.random.uniform(k1, (_B, _F)))
        u2 = np.asarray(jax.random.uniform(k2, (_B, _F)))
    mask = u1 > np.float32(1.0 - _SWAP_RATE)
    l1 = np.floor(u2 * np.float32(_B)).astype(np.int64)
    n = _B * _F
    res = (l1 * (mask.astype(np.int64) * _F)).reshape(-1)
    idx = np.arange(n, dtype=np.int64) + res
    idx = np.where(idx >= n, idx - n, idx)
    pos_p = (np.arange(n, dtype=np.int64) // _F) * _FP + np.arange(n) % _F
    src_p = (idx // _F) * _FP + idx % _F
    delta = idx != np.arange(n, dtype=np.int64)
    src = np.empty((_NW, _KMAX), np.int32)
    dst = np.empty((_NW, _KMAX), np.int32)
    w_of = pos_p // _PER_W
    loc_of = pos_p % _PER_W
    full_src = np.arange(_NP, dtype=np.int64)
    full_src[pos_p] = src_p
    for w in range(_NW):
        sel = delta & (w_of == w)
        loc = loc_of[sel]
        s = src_p[sel]
        k = loc.size
        assert k <= _KMAX
        d = np.empty(_KMAX, np.int32)
        sfull = np.empty(_KMAX, np.int32)
        d[:k] = loc
        sfull[:k] = s
        npad = _KMAX - k
        t = np.arange(npad, dtype=np.int64) * 151 % (_ROWS_W * _F)
        pad_loc = (t // _F) * _FP + t % _F
        d[k:] = pad_loc
        sfull[k:] = full_src[w * _PER_W + pad_loc]
        dst[w] = d
        src[w] = sfull
    return src, dst


_SRC_TAB, _DST_TAB = _swap_tables()


@functools.lru_cache(maxsize=None)
def _make_sc_kernel():
    info = plsc.get_sparse_core_info()
    assert info.num_cores * info.num_subcores == _NW
    mesh = plsc.VectorSubcoreMesh(core_axis_name="c", subcore_axis_name="s")

    @functools.partial(
        pl.kernel,
        mesh=mesh,
        out_type=jax.ShapeDtypeStruct((_B, _F), jnp.float32),
        compiler_params=pltpu.CompilerParams(
            needs_layout_passes=False, use_tc_tiling_on_sc=True),
        scratch_types=[
            pltpu.VMEM((_ROWS_W, _F), jnp.float32),
            pltpu.VMEM((_KMAX,), jnp.int32),
            pltpu.VMEM((_KMAX,), jnp.int32),
            pltpu.VMEM((_KMAX,), jnp.float32),
            pltpu.SemaphoreType.DMA,
            pltpu.SemaphoreType.DMA,
        ],
    )
    def swap_k(x2_hbm, xf_hbm, src_hbm, dst_hbm, out_hbm, xv, srcv, dstv,
               valsv, sem0, sem1):
        wid = lax.axis_index("s") * info.num_cores + lax.axis_index("c")
        row0 = wid * _ROWS_W
        cp_x = pltpu.async_copy(x2_hbm.at[pl.ds(row0, _ROWS_W)], xv, sem0)
        pltpu.sync_copy(src_hbm.at[wid], srcv)
        cp_g = pltpu.async_copy(xf_hbm.at[srcv], valsv, sem1)
        pltpu.sync_copy(dst_hbm.at[wid], dstv)
        cp_x.wait()
        cp_g.wait()

        def body(k, carry):
            s = pl.ds(k * 16, 16)
            d = dstv[s]
            rows = lax.shift_right_logical(d, 7)
            cols = lax.bitwise_and(d, 127)
            plsc.store_scatter(xv, [rows, cols], valsv[s])
            return carry

        lax.fori_loop(0, _KMAX // 16, body, 0)
        pltpu.sync_copy(xv, out_hbm.at[pl.ds(row0, _ROWS_W)])

    return swap_k


def kernel(x):
    xp = jnp.pad(x, ((0, 0), (0, _FP - _F)))
    return _make_sc_kernel()(
        x, xp.reshape(-1), jnp.asarray(_SRC_TAB), jnp.asarray(_DST_TAB))


# speedup vs baseline: 1.0193x; 1.0050x over previous
import contextlib
import functools

import numpy as np

import jax
import jax.numpy as jnp
from jax import lax
from jax.experimental import pallas as pl
from jax.experimental.pallas import tpu as pltpu
from jax.experimental.pallas import tpu_sc as plsc

_SWAP_RATE = 0.15
_B, _F = 16384, 100
_FP = 128
_NP = _B * _FP
_NW = 32
_ROWS_W = _B // _NW
_PER_W = _ROWS_W * _FP
_KMAX = 7840


@functools.lru_cache(maxsize=None)
def _swap_tables():
    try:
        dev = jax.local_devices(backend="cpu")[0]
        ctx = jax.default_device(dev)
    except Exception:
        ctx = contextlib.nullcontext()
    with ctx:
        k1, k2 = jax.random.split(jax.random.key(42))
        u1 = np.asarray(jax.random.uniform(k1, (_B, _F)))
        u2 = np.asarray(jax.random.uniform(k2, (_B, _F)))
    mask = u1 > np.float32(1.0 - _SWAP_RATE)
    l1 = np.floor(u2 * np.float32(_B)).astype(np.int64)
    n = _B * _F
    res = (l1 * (mask.astype(np.int64) * _F)).reshape(-1)
    idx = np.arange(n, dtype=np.int64) + res
    idx = np.where(idx >= n, idx - n, idx)
    pos_p = (np.arange(n, dtype=np.int64) // _F) * _FP + np.arange(n) % _F
    src_p = (idx // _F) * _FP + idx % _F
    delta = idx != np.arange(n, dtype=np.int64)
    src = np.empty((_NW, _KMAX), np.int32)
    dst = np.empty((_NW, _KMAX), np.int32)
    w_of = pos_p // _PER_W
    loc_of = pos_p % _PER_W
    full_src = np.arange(_NP, dtype=np.int64)
    full_src[pos_p] = src_p
    for w in range(_NW):
        sel = delta & (w_of == w)
        loc = loc_of[sel]
        s = src_p[sel]
        k = loc.size
        assert k <= _KMAX
        d = np.empty(_KMAX, np.int32)
        sfull = np.empty(_KMAX, np.int32)
        d[:k] = loc
        sfull[:k] = s
        npad = _KMAX - k
        t = np.arange(npad, dtype=np.int64) * 151 % (_ROWS_W * _F)
        pad_loc = (t // _F) * _FP + t % _F
        d[k:] = pad_loc
        sfull[k:] = full_src[w * _PER_W + pad_loc]
        dst[w] = d
        src[w] = sfull
    return src, dst


_SRC_TAB, _DST_TAB = _swap_tables()


@functools.lru_cache(maxsize=None)
def _make_sc_kernel():
    info = plsc.get_sparse_core_info()
    assert info.num_cores * info.num_subcores == _NW
    mesh = plsc.VectorSubcoreMesh(core_axis_name="c", subcore_axis_name="s")

    @functools.partial(
        pl.kernel,
        mesh=mesh,
        out_type=jax.ShapeDtypeStruct((_B, _F), jnp.float32),
        compiler_params=pltpu.CompilerParams(
            needs_layout_passes=False, use_tc_tiling_on_sc=True),
        scratch_types=[
            pltpu.VMEM((_ROWS_W, _F), jnp.float32),
            pltpu.VMEM((_KMAX,), jnp.int32),
            pltpu.VMEM((_KMAX,), jnp.int32),
            pltpu.VMEM((_KMAX,), jnp.float32),
            pltpu.SemaphoreType.DMA,
            pltpu.SemaphoreType.DMA,
        ],
    )
    def swap_k(x2_hbm, xf_hbm, src_hbm, dst_hbm, out_hbm, xv, srcv, dstv,
               valsv, sem0, sem1):
        wid = lax.axis_index("s") * info.num_cores + lax.axis_index("c")
        row0 = wid * _ROWS_W
        cp_x = pltpu.async_copy(x2_hbm.at[pl.ds(row0, _ROWS_W)], xv, sem0)
        pltpu.sync_copy(src_hbm.at[wid], srcv)
        cp_g = pltpu.async_copy(xf_hbm.at[srcv], valsv, sem1)
        pltpu.sync_copy(dst_hbm.at[wid], dstv)
        cp_x.wait()
        cp_g.wait()

        def body(k, carry):
            s = pl.ds(k * 16, 16)
            d = dstv[s]
            rows = lax.shift_right_logical(d, 7)
            cols = lax.bitwise_and(d, 127)
            plsc.store_scatter(xv, [rows, cols], valsv[s])
            return carry

        lax.fori_loop(0, _KMAX // 16, body, 0, unroll=4)
        pltpu.sync_copy(xv, out_hbm.at[pl.ds(row0, _ROWS_W)])

    return swap_k


def kernel(x):
    xp = jnp.pad(x, ((0, 0), (0, _FP - _F)))
    return _make_sc_kernel()(
        x, xp.reshape(-1), jnp.asarray(_SRC_TAB), jnp.asarray(_DST_TAB))


# speedup vs baseline: 1.0220x; 1.0027x over previous
import contextlib
import functools

import numpy as np

import jax
import jax.numpy as jnp
from jax import lax
from jax.experimental import pallas as pl
from jax.experimental.pallas import tpu as pltpu
from jax.experimental.pallas import tpu_sc as plsc

_SWAP_RATE = 0.15
_B, _F = 16384, 100
_FP = 128
_NP = _B * _FP
_NW = 32
_ROWS_W = _B // _NW
_PER_W = _ROWS_W * _FP
_KMAX = 7840


@functools.lru_cache(maxsize=None)
def _swap_tables():
    try:
        dev = jax.local_devices(backend="cpu")[0]
        ctx = jax.default_device(dev)
    except Exception:
        ctx = contextlib.nullcontext()
    with ctx:
        k1, k2 = jax.random.split(jax.random.key(42))
        u1 = np.asarray(jax.random.uniform(k1, (_B, _F)))
        u2 = np.asarray(jax.random.uniform(k2, (_B, _F)))
    mask = u1 > np.float32(1.0 - _SWAP_RATE)
    l1 = np.floor(u2 * np.float32(_B)).astype(np.int64)
    n = _B * _F
    res = (l1 * (mask.astype(np.int64) * _F)).reshape(-1)
    idx = np.arange(n, dtype=np.int64) + res
    idx = np.where(idx >= n, idx - n, idx)
    pos_p = (np.arange(n, dtype=np.int64) // _F) * _FP + np.arange(n) % _F
    src_p = (idx // _F) * _FP + idx % _F
    delta = idx != np.arange(n, dtype=np.int64)
    src = np.empty((_NW, _KMAX), np.int32)
    dst = np.empty((_NW, _KMAX), np.int32)
    w_of = pos_p // _PER_W
    loc_of = pos_p % _PER_W
    full_src = np.arange(_NP, dtype=np.int64)
    full_src[pos_p] = src_p
    for w in range(_NW):
        sel = delta & (w_of == w)
        loc = loc_of[sel]
        s = src_p[sel]
        k = loc.size
        assert k <= _KMAX
        d = np.empty(_KMAX, np.int32)
        sfull = np.empty(_KMAX, np.int32)
        d[:k] = loc
        sfull[:k] = s
        npad = _KMAX - k
        t = np.arange(npad, dtype=np.int64) * 151 % (_ROWS_W * _F)
        pad_loc = (t // _F) * _FP + t % _F
        d[k:] = pad_loc
        sfull[k:] = full_src[w * _PER_W + pad_loc]
        dst[w] = d
        src[w] = sfull
    return src, dst


_SRC_TAB, _DST_TAB = _swap_tables()


@functools.lru_cache(maxsize=None)
def _make_sc_kernel():
    info = plsc.get_sparse_core_info()
    assert info.num_cores * info.num_subcores == _NW
    mesh = plsc.VectorSubcoreMesh(core_axis_name="c", subcore_axis_name="s")

    @functools.partial(
        pl.kernel,
        mesh=mesh,
        out_type=jax.ShapeDtypeStruct((_B, _F), jnp.float32),
        compiler_params=pltpu.CompilerParams(
            needs_layout_passes=False, use_tc_tiling_on_sc=True),
        scratch_types=[
            pltpu.VMEM((_ROWS_W, _F), jnp.float32),
            pltpu.VMEM((_KMAX,), jnp.int32),
            pltpu.VMEM((_KMAX,), jnp.int32),
            pltpu.VMEM((_KMAX,), jnp.float32),
            pltpu.SemaphoreType.DMA,
            pltpu.SemaphoreType.DMA,
        ],
    )
    def swap_k(x2_hbm, xf_hbm, src_hbm, dst_hbm, out_hbm, xv, srcv, dstv,
               valsv, sem0, sem1):
        wid = lax.axis_index("s") * info.num_cores + lax.axis_index("c")
        row0 = wid * _ROWS_W
        cp_x = pltpu.async_copy(x2_hbm.at[pl.ds(row0, _ROWS_W)], xv, sem0)
        pltpu.sync_copy(src_hbm.at[wid], srcv)
        cp_g = pltpu.async_copy(xf_hbm.at[srcv], valsv, sem1)
        pltpu.sync_copy(dst_hbm.at[wid], dstv)
        cp_x.wait()
        cp_g.wait()

        def body(k, carry):
            s = pl.ds(k * 16, 16)
            d = dstv[s]
            rows = lax.shift_right_logical(d, 7)
            cols = lax.bitwise_and(d, 127)
            plsc.store_scatter(xv, [rows, cols], valsv[s])
            return carry

        lax.fori_loop(0, _KMAX // 16, body, 0, unroll=8)
        pltpu.sync_copy(xv, out_hbm.at[pl.ds(row0, _ROWS_W)])

    return swap_k


def kernel(x):
    xp = jnp.pad(x, ((0, 0), (0, _FP - _F)))
    return _make_sc_kernel()(
        x, xp.reshape(-1), jnp.asarray(_SRC_TAB), jnp.asarray(_DST_TAB))


# speedup vs baseline: 1.0277x; 1.0055x over previous
import contextlib
import functools

import numpy as np

import jax
import jax.numpy as jnp
from jax import lax
from jax.experimental import pallas as pl
from jax.experimental.pallas import tpu as pltpu
from jax.experimental.pallas import tpu_sc as plsc

_SWAP_RATE = 0.15
_B, _F = 16384, 100
_FP = 128
_NP = _B * _FP
_NW = 32
_ROWS_W = _B // _NW
_PER_W = _ROWS_W * _FP
_KMAX = 7840


@functools.lru_cache(maxsize=None)
def _swap_tables():
    try:
        dev = jax.local_devices(backend="cpu")[0]
        ctx = jax.default_device(dev)
    except Exception:
        ctx = contextlib.nullcontext()
    with ctx:
        k1, k2 = jax.random.split(jax.random.key(42))
        u1 = np.asarray(jax.random.uniform(k1, (_B, _F)))
        u2 = np.asarray(jax.random.uniform(k2, (_B, _F)))
    mask = u1 > np.float32(1.0 - _SWAP_RATE)
    l1 = np.floor(u2 * np.float32(_B)).astype(np.int64)
    n = _B * _F
    res = (l1 * (mask.astype(np.int64) * _F)).reshape(-1)
    idx = np.arange(n, dtype=np.int64) + res
    idx = np.where(idx >= n, idx - n, idx)
    pos_p = (np.arange(n, dtype=np.int64) // _F) * _FP + np.arange(n) % _F
    src_p = (idx // _F) * _FP + idx % _F
    delta = idx != np.arange(n, dtype=np.int64)
    src = np.empty((_NW, _KMAX), np.int32)
    dst = np.empty((_NW, _KMAX), np.int32)
    w_of = pos_p // _PER_W
    loc_of = pos_p % _PER_W
    full_src = np.arange(_NP, dtype=np.int64)
    full_src[pos_p] = src_p
    for w in range(_NW):
        sel = delta & (w_of == w)
        loc = loc_of[sel]
        s = src_p[sel]
        k = loc.size
        assert k <= _KMAX
        d = np.empty(_KMAX, np.int32)
        sfull = np.empty(_KMAX, np.int32)
        d[:k] = loc
        sfull[:k] = s
        npad = _KMAX - k
        t = np.arange(npad, dtype=np.int64) * 151 % (_ROWS_W * _F)
        pad_loc = (t // _F) * _FP + t % _F
        d[k:] = pad_loc
        sfull[k:] = full_src[w * _PER_W + pad_loc]
        dst[w] = d
        src[w] = sfull
    return src, dst


_SRC_TAB, _DST_TAB = _swap_tables()


@functools.lru_cache(maxsize=None)
def _make_sc_kernel():
    info = plsc.get_sparse_core_info()
    assert info.num_cores * info.num_subcores == _NW
    mesh = plsc.VectorSubcoreMesh(core_axis_name="c", subcore_axis_name="s")

    @functools.partial(
        pl.kernel,
        mesh=mesh,
        out_type=jax.ShapeDtypeStruct((_B, _F), jnp.float32),
        compiler_params=pltpu.CompilerParams(
            needs_layout_passes=False, use_tc_tiling_on_sc=True),
        scratch_types=[
            pltpu.VMEM((_ROWS_W, _F), jnp.float32),
            pltpu.VMEM((_KMAX,), jnp.int32),
            pltpu.VMEM((_KMAX,), jnp.int32),
            pltpu.VMEM((_KMAX,), jnp.float32),
            pltpu.SemaphoreType.DMA,
            pltpu.SemaphoreType.DMA,
            pltpu.SemaphoreType.DMA,
        ],
    )
    def swap_k(x2_hbm, xf_hbm, src_hbm, dst_hbm, out_hbm, xv, srcv, dstv,
               valsv, sem0, sem1, sem2):
        wid = lax.axis_index("s") * info.num_cores + lax.axis_index("c")
        row0 = wid * _ROWS_W
        half = _KMAX // 2
        cp_x = pltpu.async_copy(x2_hbm.at[pl.ds(row0, _ROWS_W)], xv, sem0)
        pltpu.sync_copy(src_hbm.at[wid], srcv)
        cp_g = pltpu.async_copy(
            xf_hbm.at[srcv.at[pl.ds(0, half)]], valsv.at[pl.ds(0, half)],
            sem1)
        cp_g2 = pltpu.async_copy(
            xf_hbm.at[srcv.at[pl.ds(half, half)]],
            valsv.at[pl.ds(half, half)], sem2)
        pltpu.sync_copy(dst_hbm.at[wid], dstv)
        cp_x.wait()
        cp_g.wait()
        cp_g2.wait()

        def body(k, carry):
            s = pl.ds(k * 16, 16)
            d = dstv[s]
            rows = lax.shift_right_logical(d, 7)
            cols = lax.bitwise_and(d, 127)
            plsc.store_scatter(xv, [rows, cols], valsv[s])
            return carry

        lax.fori_loop(0, _KMAX // 16, body, 0, unroll=8)
        pltpu.sync_copy(xv, out_hbm.at[pl.ds(row0, _ROWS_W)])

    return swap_k


def kernel(x):
    xp = jnp.pad(x, ((0, 0), (0, _FP - _F)))
    return _make_sc_kernel()(
        x, xp.reshape(-1), jnp.asarray(_SRC_TAB), jnp.asarray(_DST_TAB))
